# Initial kernel scaffold; baseline (speedup 1.0000x reference)
import jax
import jax.numpy as jnp
from jax.experimental import pallas as pl


def kernel(x, edge_index, weight, W1, b1, W2, b2, W3, b3, W4):
    raise NotImplementedError("write your pallas kernel here")


# speedup vs baseline: 1.7925x; 1.7925x over previous
import functools

import jax
import jax.numpy as jnp
from jax import lax
from jax.experimental import pallas as pl
from jax.experimental.pallas import tpu as pltpu
from jax.experimental.pallas import tpu_sc as plsc

NN = 10000
NP = 10240
NC = 2
NS = 16
NW = NC * NS
RPW = NP // NW
EE = 160000
CHP = 1024
EP = -(-(EE + NW * CHP) // 512) * 512
EPW = EP // NW


def _norm_body(rows_hbm, cols_hbm, w_hbm, dis_hbm, norm_hbm,
               rbuf, cbuf, wbuf, disfull, nbuf):
    w = lax.axis_index("s") * NC + lax.axis_index("c")
    t0 = w * EPW
    pltpu.sync_copy(rows_hbm.at[pl.ds(t0, EPW)], rbuf)
    pltpu.sync_copy(cols_hbm.at[pl.ds(t0, EPW)], cbuf)
    pltpu.sync_copy(w_hbm.at[pl.ds(t0, EPW)], wbuf)
    pltpu.sync_copy(dis_hbm, disfull)

    def nmbody(g, _):
        rv = rbuf[pl.ds(g * 16, 16)]
        cv = cbuf[pl.ds(g * 16, 16)]
        wv = wbuf[pl.ds(g * 16, 16)]
        wm = jnp.where(rv == cv, 0.0, wv)
        dr = plsc.load_gather(disfull, [rv])
        dc = plsc.load_gather(disfull, [cv])
        nbuf[pl.ds(g * 16, 16)] = -((dr * wm) * dc)
        return 0
    lax.fori_loop(0, EPW // 16, nmbody, 0)
    pltpu.sync_copy(nbuf, norm_hbm.at[pl.ds(t0, EPW)])


@functools.cache
def _norm_kernel():
    mesh = plsc.VectorSubcoreMesh(core_axis_name="c", subcore_axis_name="s")
    return jax.jit(pl.kernel(
        _norm_body,
        out_type=jax.ShapeDtypeStruct((EP,), jnp.float32),
        mesh=mesh,
        scratch_types=[
            pltpu.VMEM((EPW,), jnp.int32),
            pltpu.VMEM((EPW,), jnp.int32),
            pltpu.VMEM((EPW,), jnp.float32),
            pltpu.VMEM((NP,), jnp.float32),
            pltpu.VMEM((EPW,), jnp.float32),
        ],
        compiler_params=pltpu.CompilerParams(
            needs_layout_passes=False, use_tc_tiling_on_sc=False),
    ))


def _make_prop_body(fp, beta, has_prev, ch, pb):
    nv = fp // 16
    kch = ch // 128

    def body(*refs):
        t_hbm, rows_hbm, cols_hbm, norm_hbm, prm_hbm = refs[:5]
        i = 5
        prev_hbm = None
        if has_prev:
            prev_hbm = refs[i]; i += 1
        out_hbm = refs[i]; i += 1
        pbuf, rowbuf, colbuf, nbuf, gbuf, acc, gsem = refs[i:i + 7]
        i += 7
        prevbuf = refs[i] if has_prev else None

        w = lax.axis_index("s") * NC + lax.axis_index("c")
        pltpu.sync_copy(prm_hbm.at[w], pbuf)
        pv = pbuf[0, pl.ds(0, 16)]
        estart = pv[0]
        nch = pv[1] * (CHP // ch)

        zero16 = jnp.zeros((16,), jnp.float32)

        def zbody(r, _):
            for v in range(nv):
                acc[r, pl.ds(v * 16, 16)] = zero16
            return 0
        lax.fori_loop(0, RPW, zbody, 0)

        def chunk(c, _):
            base = pl.multiple_of(estart + c * ch, 8)
            cb = (estart >> 7) + c * kch
            for k in range(kch):
                pltpu.sync_copy(rows_hbm.at[cb + k], rowbuf.at[pl.ds(k, 1)])
            pltpu.sync_copy(cols_hbm.at[pl.ds(base, ch)], colbuf)
            pltpu.sync_copy(norm_hbm.at[pl.ds(base, ch)], nbuf)
            handles = []
            for k in range(kch):
                handles.append(pltpu.async_copy(
                    t_hbm.at[rowbuf.at[k]], gbuf.at[pl.ds(k * 128, 128)],
                    gsem))
            for h in handles:
                h.wait()

            def grp(g, _):
                cv = colbuf[pl.ds(g * 16, 16)]
                nvec = nbuf[pl.ds(g * 16, 16)]
                for j in range(16):
                    cj = cv[j]
                    nj = nvec[j]
                    e = g * 16 + j
                    for v in range(nv):
                        val = gbuf[e, pl.ds(v * 16, 16)]
                        acc[cj, pl.ds(v * 16, 16)] = (
                            acc[cj, pl.ds(v * 16, 16)] + val * nj)
                return 0
            lax.fori_loop(0, ch // 16, grp, 0)
            return 0
        lax.fori_loop(0, nch, chunk, 0)

        n0 = w * RPW
        for s in range(RPW // pb if (has_prev or beta != 1.0) else 0):
            if has_prev:
                pltpu.sync_copy(prev_hbm.at[pl.ds(n0 + s * pb, pb)], prevbuf)

            def comb(r, _):
                for v in range(nv):
                    o = acc[s * pb + r, pl.ds(v * 16, 16)]
                    if beta != 1.0:
                        o = o * beta
                    if has_prev:
                        o = o - prevbuf[r, pl.ds(v * 16, 16)]
                    acc[s * pb + r, pl.ds(v * 16, 16)] = o
                return 0
            lax.fori_loop(0, pb, comb, 0)
        pltpu.sync_copy(acc, out_hbm.at[pl.ds(n0, RPW)])

    return body


@functools.cache
def _prop_kernel(fp, beta, has_prev):
    ch = 256 if fp > 32 else 1024
    pb = 80 if fp > 32 else RPW
    scratch = [
        pltpu.VMEM((1, 16), jnp.int32),
        pltpu.VMEM((ch // 128, 128), jnp.int32),
        pltpu.VMEM((ch,), jnp.int32),
        pltpu.VMEM((ch,), jnp.float32),
        pltpu.VMEM((ch, fp), jnp.float32),
        pltpu.VMEM((RPW, fp), jnp.float32),
        pltpu.SemaphoreType.DMA,
    ]
    if has_prev:
        scratch.append(pltpu.VMEM((pb, fp), jnp.float32))
    mesh = plsc.VectorSubcoreMesh(core_axis_name="c", subcore_axis_name="s")
    return jax.jit(pl.kernel(
        _make_prop_body(fp, beta, has_prev, ch, pb),
        out_type=jax.ShapeDtypeStruct((NP, fp), jnp.float32),
        mesh=mesh,
        scratch_types=scratch,
        compiler_params=pltpu.CompilerParams(
            needs_layout_passes=False, use_tc_tiling_on_sc=False),
    ))


def _cheb_mm_body(x_ref, w_ref, o_ref):
    kk = x_ref.shape[0]
    o = jnp.dot(x_ref[0], w_ref[0], preferred_element_type=jnp.float32)
    for k in range(1, kk):
        o = o + jnp.dot(x_ref[k], w_ref[k],
                        preferred_element_type=jnp.float32)
    o_ref[...] = o


def _tc_cheb(stack, wstack, bm):
    kk, m, fi = stack.shape
    fo = wstack.shape[2]
    return pl.pallas_call(
        _cheb_mm_body,
        grid=(m // bm,),
        in_specs=[pl.BlockSpec((kk, bm, fi), lambda i: (0, i, 0)),
                  pl.BlockSpec((kk, fi, fo), lambda i: (0, 0, 0))],
        out_specs=pl.BlockSpec((bm, fo), lambda i: (i, 0)),
        out_shape=jax.ShapeDtypeStruct((m, fo), jnp.float32),
    )(stack, wstack)


def _edge_layout(row, col, weight):
    perm = jnp.argsort(col, stable=True)
    row_s = jnp.take(row, perm)
    col_s = jnp.take(col, perm)
    w_s = jnp.take(weight, perm)
    qb = jnp.arange(NW + 1, dtype=jnp.int32) * RPW
    bnd = jnp.searchsorted(col_s, qb).astype(jnp.int32)
    cnt = bnd[1:] - bnd[:-1]
    nch = (cnt + (CHP - 1)) // CHP
    off = jnp.concatenate([jnp.zeros((1,), jnp.int32),
                           jnp.cumsum(nch * CHP).astype(jnp.int32)])
    ar = jnp.arange(EP, dtype=jnp.int32)
    ws = jnp.clip(jnp.searchsorted(off, ar, side="right").astype(jnp.int32) - 1,
                  0, NW - 1)
    eidx = ar - jnp.take(off, ws) + jnp.take(bnd, ws)
    valid = eidx < jnp.take(bnd, ws + 1)
    ec = jnp.clip(eidx, 0, EE - 1)
    row_p = jnp.where(valid, jnp.take(row_s, ec), 0)
    colg_p = jnp.where(valid, jnp.take(col_s, ec), 0)
    w_p = jnp.where(valid, jnp.take(w_s, ec), 0.0)
    coll_p = jnp.where(valid, colg_p - ws * RPW, 0)
    prm = jnp.zeros((NW, 16), jnp.int32)
    prm = prm.at[:, 0].set(off[:-1]).at[:, 1].set(nch)
    return row_p, colg_p, coll_p, w_p, prm.reshape(NW, 1, 16)


def _cheb_layer(h, fp, k_order, rows3d, coll_p, norm_p, prm):
    first = _prop_kernel(fp, 1.0, False)
    rec = _prop_kernel(fp, 2.0, True)
    txs = [h]
    if k_order > 1:
        txs.append(first(h, rows3d, coll_p, norm_p, prm))
    for _ in range(2, k_order):
        txs.append(rec(txs[-1], rows3d, coll_p, norm_p, prm, txs[-2]))
    return jnp.stack(txs)


def kernel(x, edge_index, weight, W1, b1, W2, b2, W3, b3, W4):
    row = edge_index[0]
    col = edge_index[1]
    row_p, colg_p, coll_p, w_p, prm = _edge_layout(row, col, weight)
    rows3d = row_p.reshape(EP // 128, 1, 128)

    wm = jnp.where(row == col, jnp.zeros_like(weight), weight)
    deg = jnp.zeros((NN,), weight.dtype).at[row].add(wm)
    dis = jnp.where(deg > 0, lax.rsqrt(jnp.maximum(deg, 1e-12)), 0.0)
    disp = jnp.pad(dis, (0, NP - NN))
    norm_p = _norm_kernel()(row_p, colg_p, w_p, disp)

    xp = jnp.pad(x, ((0, NP - NN), (0, 0)))
    s1 = _cheb_layer(xp, 128, W1.shape[0], rows3d, coll_p, norm_p, prm)
    w1s = jnp.pad(W1, ((0, 0), (0, 0), (0, 2)))
    h1 = jax.nn.silu(_tc_cheb(s1, w1s, 256) + jnp.pad(b1, (0, 2)))

    s2 = _cheb_layer(h1, 16, W2.shape[0], rows3d, coll_p, norm_p, prm)
    w2s = jnp.pad(W2, ((0, 0), (0, 2), (0, 12)))
    h2 = jax.nn.silu(_tc_cheb(s2, w2s, 1024) + jnp.pad(b2, (0, 12)))

    s3 = _cheb_layer(h2, 32, W3.shape[0], rows3d, coll_p, norm_p, prm)
    w3s = jnp.pad(W3, ((0, 0), (0, 12), (0, 5)))
    h3 = jax.nn.silu(_tc_cheb(s3, w3s, 512) + jnp.pad(b3, (0, 5)))

    w4s = jnp.pad(W4, ((0, 0), (0, 5), (0, 0)))
    out = jax.nn.sigmoid(_tc_cheb(h3[None], w4s, 1024))
    return out[:NN]
